# baseline (speedup 1.0000x reference)
import functools

import jax
import jax.numpy as jnp
from jax import lax
from jax.experimental import pallas as pl
from jax.experimental.pallas import tpu as pltpu
from jax.experimental.pallas import tpu_sc as plsc

L = 16
NC = 2
NS = 16
NW = NC * NS
V = 10

B, SEQ = 16384, 200
COLS_W = B // NW
RCHUNK = 40
SEQ_SC = 160
NCHUNK = SEQ_SC // RCHUNK
VROW = COLS_W // L
PAIRS = VROW // 2
TC_BR = (SEQ - SEQ_SC) // 8
TC_R0 = SEQ_SC // 8


@functools.partial(
    pl.kernel,
    out_type=jax.ShapeDtypeStruct((NW, L), jnp.float32),
    mesh=plsc.VectorSubcoreMesh(core_axis_name="c", subcore_axis_name="s"),
    compiler_params=pltpu.CompilerParams(needs_layout_passes=False),
    scratch_types=[
        pltpu.VMEM((RCHUNK, COLS_W), jnp.int32),
        pltpu.VMEM((RCHUNK, COLS_W), jnp.int32),
        pltpu.VMEM((V, V), jnp.float32),
        pltpu.VMEM((V, V), jnp.float32),
        pltpu.VMEM((L * L,), jnp.float32),
        pltpu.VMEM((1, L), jnp.float32),
        pltpu.SemaphoreType.DMA,
        pltpu.SemaphoreType.DMA,
    ],
)
def _sc_sum(xt_hbm, ea_hbm, eb_hbm, out_hbm,
            buf0, buf1, tab_a, tab_b, pair, acc_ref, sem0, sem1):
    cid = lax.axis_index("c")
    sid = lax.axis_index("s")
    wid = sid * NC + cid
    col0 = wid * COLS_W

    pltpu.sync_copy(ea_hbm, tab_a)
    pltpu.sync_copy(eb_hbm, tab_b)
    rows = lax.iota(jnp.int32, L)
    keep = rows < V
    zeros = jnp.zeros((L,), jnp.float32)
    s = zeros
    for k in range(V):
        col = jnp.full((L,), k, jnp.int32)
        s = s + plsc.load_gather(tab_a, [rows, col], mask=keep)
        s = s + plsc.load_gather(tab_b, [rows, col], mask=keep)
    s = jnp.where(keep, s, zeros)

    for a in range(V):
        pair[pl.ds(a * L, L)] = s[a] + s

    bufs = (buf0, buf1)
    sems = (sem0, sem1)

    def dma(c, buf, sem):
        return pltpu.make_async_copy(
            xt_hbm.at[pl.ds(c * RCHUNK, RCHUNK), pl.ds(col0, COLS_W)],
            buf, sem)

    dma(0, buf0, sem0).start()
    acc = zeros
    for c in range(NCHUNK):
        buf, sem = bufs[c % 2], sems[c % 2]
        if c + 1 < NCHUNK:
            dma(c + 1, bufs[(c + 1) % 2], sems[(c + 1) % 2]).start()
        dma(c, buf, sem).wait()

        def body(r, a, buf=buf):
            for j in range(PAIRS):
                ia = buf[r, pl.ds(2 * j * L, L)]
                ib = buf[r, pl.ds((2 * j + 1) * L, L)]
                a = a + plsc.load_gather(pair, [ia * L + ib])
            return a

        acc = lax.fori_loop(0, RCHUNK, body, acc)

    acc_ref[0, :] = acc
    pltpu.sync_copy(acc_ref, out_hbm.at[pl.ds(wid, 1)])


def _tc_body(x_ref, ea_ref, eb_ref, out_ref, s_ref):
    r = pl.program_id(0)
    c = pl.program_id(1)

    @pl.when((r == 0) & (c == 0))
    def _init():
        out_ref[0, 0] = 0.0
        for v in range(V):
            s_ref[v] = jnp.sum(ea_ref[v, :]) + jnp.sum(eb_ref[v, :])

    xb = x_ref[...]
    val = jnp.full(xb.shape, s_ref[V - 1], jnp.float32)
    for v in range(V - 2, -1, -1):
        val = jnp.where(xb == v, s_ref[v], val)
    out_ref[0, 0] += jnp.sum(val)


_tc_sum = pl.pallas_call(
    _tc_body,
    grid=(TC_BR, 8),
    in_specs=[
        pl.BlockSpec((8, B // 8), lambda r, c: (TC_R0 + r, c)),
        pl.BlockSpec((V, V), lambda r, c: (0, 0)),
        pl.BlockSpec((V, V), lambda r, c: (0, 0)),
    ],
    out_specs=pl.BlockSpec(memory_space=pltpu.SMEM),
    out_shape=jax.ShapeDtypeStruct((1, 1), jnp.float32),
    scratch_shapes=[pltpu.SMEM((V,), jnp.float32)],
)


def kernel(x, emb, emb2):
    xt = x.astype(jnp.int32).T
    partials = _sc_sum(xt, emb, emb2)
    tc_part = _tc_sum(xt, emb, emb2)
    return jnp.sum(partials) + tc_part[0, 0]


# speedup vs baseline: 1.0548x; 1.0548x over previous
import functools

import jax
import jax.numpy as jnp
from jax import lax
from jax.experimental import pallas as pl
from jax.experimental.pallas import tpu as pltpu
from jax.experimental.pallas import tpu_sc as plsc

L = 16
NC = 2
NS = 16
NW = NC * NS
V = 10

B, SEQ = 16384, 200
COLS_W = B // NW
RCHUNK = 40
SEQ_SC = 160
NCHUNK = SEQ_SC // RCHUNK
VROW = COLS_W // L
PAIRS = VROW // 2
TC_BR = (SEQ - SEQ_SC) // 8
TC_R0 = SEQ_SC // 8


@functools.partial(
    pl.kernel,
    out_type=jax.ShapeDtypeStruct((NW, L), jnp.float32),
    mesh=plsc.VectorSubcoreMesh(core_axis_name="c", subcore_axis_name="s"),
    compiler_params=pltpu.CompilerParams(needs_layout_passes=False),
    scratch_types=[
        pltpu.VMEM((RCHUNK, COLS_W), jnp.int32),
        pltpu.VMEM((RCHUNK, COLS_W), jnp.int32),
        pltpu.VMEM((V, V), jnp.float32),
        pltpu.VMEM((V, V), jnp.float32),
        pltpu.VMEM((L * L,), jnp.float32),
        pltpu.VMEM((1, L), jnp.float32),
        pltpu.SemaphoreType.DMA,
        pltpu.SemaphoreType.DMA,
    ],
)
def _sc_sum(xt_hbm, ea_hbm, eb_hbm, out_hbm,
            buf0, buf1, tab_a, tab_b, pair, acc_ref, sem0, sem1):
    cid = lax.axis_index("c")
    sid = lax.axis_index("s")
    wid = sid * NC + cid
    col0 = wid * COLS_W

    pltpu.sync_copy(ea_hbm, tab_a)
    pltpu.sync_copy(eb_hbm, tab_b)
    rows = lax.iota(jnp.int32, L)
    keep = rows < V
    zeros = jnp.zeros((L,), jnp.float32)
    s = zeros
    for k in range(V):
        col = jnp.full((L,), k, jnp.int32)
        s = s + plsc.load_gather(tab_a, [rows, col], mask=keep)
        s = s + plsc.load_gather(tab_b, [rows, col], mask=keep)
    s = jnp.where(keep, s, zeros)

    for a in range(V):
        pair[pl.ds(a * L, L)] = s[a] + s

    bufs = (buf0, buf1)
    sems = (sem0, sem1)

    def dma(c, buf, sem):
        return pltpu.make_async_copy(
            xt_hbm.at[pl.ds(c * RCHUNK, RCHUNK), pl.ds(col0, COLS_W)],
            buf, sem)

    dma(0, buf0, sem0).start()
    acc = zeros
    for c in range(NCHUNK):
        buf, sem = bufs[c % 2], sems[c % 2]
        if c + 1 < NCHUNK:
            dma(c + 1, bufs[(c + 1) % 2], sems[(c + 1) % 2]).start()
        dma(c, buf, sem).wait()

        def body(r, a, buf=buf):
            for j in range(PAIRS):
                ia = buf[r, pl.ds(2 * j * L, L)]
                ib = buf[r, pl.ds((2 * j + 1) * L, L)]
                a = a + plsc.load_gather(pair, [ia * L + ib])
            return a

        acc = lax.fori_loop(0, RCHUNK, body, acc)

    acc_ref[0, :] = acc
    pltpu.sync_copy(acc_ref, out_hbm.at[pl.ds(wid, 1)])


def _tc_body(x_ref, ea_ref, eb_ref, out_ref, s_ref, vacc_ref):
    r = pl.program_id(0)
    c = pl.program_id(1)

    @pl.when((r == 0) & (c == 0))
    def _init():
        vacc_ref[...] = jnp.zeros((8, 128), jnp.float32)
        for v in range(V):
            s_ref[v] = jnp.sum(ea_ref[v, :]) + jnp.sum(eb_ref[v, :])

    xb = x_ref[...]
    val = jnp.full(xb.shape, s_ref[V - 1], jnp.float32)
    for v in range(V - 2, -1, -1):
        val = jnp.where(xb == v, s_ref[v], val)
    acc = val[:, 0:128]
    for k in range(1, (B // 8) // 128):
        acc = acc + val[:, k * 128:(k + 1) * 128]
    vacc_ref[...] += acc

    @pl.when((r == TC_BR - 1) & (c == 7))
    def _fini():
        out_ref[0, 0] = jnp.sum(vacc_ref[...])


_tc_sum = pl.pallas_call(
    _tc_body,
    grid=(TC_BR, 8),
    in_specs=[
        pl.BlockSpec((8, B // 8), lambda r, c: (TC_R0 + r, c)),
        pl.BlockSpec((V, V), lambda r, c: (0, 0)),
        pl.BlockSpec((V, V), lambda r, c: (0, 0)),
    ],
    out_specs=pl.BlockSpec(memory_space=pltpu.SMEM),
    out_shape=jax.ShapeDtypeStruct((1, 1), jnp.float32),
    scratch_shapes=[pltpu.SMEM((V,), jnp.float32),
                    pltpu.VMEM((8, 128), jnp.float32)],
)


def kernel(x, emb, emb2):
    xt = x.astype(jnp.int32).T
    partials = _sc_sum(xt, emb, emb2)
    tc_part = _tc_sum(xt, emb, emb2)
    return jnp.sum(partials) + tc_part[0, 0]


# speedup vs baseline: 1.1868x; 1.1251x over previous
import functools

import jax
import jax.numpy as jnp
from jax import lax
from jax.experimental import pallas as pl
from jax.experimental.pallas import tpu as pltpu
from jax.experimental.pallas import tpu_sc as plsc

L = 16
NC = 2
NS = 16
NW = NC * NS
V = 10

B, SEQ = 16384, 200
COLS_W = B // NW
RCHUNK = 40
SEQ_SC = 160
NCHUNK = SEQ_SC // RCHUNK
VROW = COLS_W // L
PAIRS = VROW // 2
TC_BR = (SEQ - SEQ_SC) // 8
TC_R0 = SEQ_SC // 8


@functools.partial(
    pl.kernel,
    out_type=jax.ShapeDtypeStruct((NW, L), jnp.float32),
    mesh=plsc.VectorSubcoreMesh(core_axis_name="c", subcore_axis_name="s"),
    compiler_params=pltpu.CompilerParams(needs_layout_passes=False),
    scratch_types=[
        pltpu.VMEM((RCHUNK, COLS_W), jnp.int32),
        pltpu.VMEM((RCHUNK, COLS_W), jnp.int32),
        pltpu.VMEM((V, V), jnp.float32),
        pltpu.VMEM((V, V), jnp.float32),
        pltpu.VMEM((L * L,), jnp.float32),
        pltpu.VMEM((1, L), jnp.float32),
        pltpu.SemaphoreType.DMA,
        pltpu.SemaphoreType.DMA,
    ],
)
def _sc_sum(xt_hbm, ea_hbm, eb_hbm, out_hbm,
            buf0, buf1, tab_a, tab_b, pair, acc_ref, sem0, sem1):
    cid = lax.axis_index("c")
    sid = lax.axis_index("s")
    wid = sid * NC + cid
    col0 = wid * COLS_W

    pltpu.sync_copy(ea_hbm, tab_a)
    pltpu.sync_copy(eb_hbm, tab_b)
    rows = lax.iota(jnp.int32, L)
    keep = rows < V
    zeros = jnp.zeros((L,), jnp.float32)
    s = zeros
    for k in range(V):
        col = jnp.full((L,), k, jnp.int32)
        s = s + plsc.load_gather(tab_a, [rows, col], mask=keep)
        s = s + plsc.load_gather(tab_b, [rows, col], mask=keep)
    s = jnp.where(keep, s, zeros)

    for a in range(V):
        pair[pl.ds(a * L, L)] = s[a] + s

    bufs = (buf0, buf1)
    sems = (sem0, sem1)

    def dma(c, buf, sem):
        return pltpu.make_async_copy(
            xt_hbm.at[pl.ds(c * RCHUNK, RCHUNK), pl.ds(col0, COLS_W)],
            buf, sem)

    dma(0, buf0, sem0).start()
    acc = zeros
    for c in range(NCHUNK):
        buf, sem = bufs[c % 2], sems[c % 2]
        if c + 1 < NCHUNK:
            dma(c + 1, bufs[(c + 1) % 2], sems[(c + 1) % 2]).start()
        dma(c, buf, sem).wait()

        def body(r, a, buf=buf):
            for j in range(PAIRS):
                ia = buf[r, pl.ds(2 * j * L, L)]
                ib = buf[r, pl.ds((2 * j + 1) * L, L)]
                a = a + plsc.load_gather(pair, [ia * L + ib])
            return a

        acc = lax.fori_loop(0, RCHUNK, body, acc)

    acc_ref[0, :] = acc
    pltpu.sync_copy(acc_ref, out_hbm.at[pl.ds(wid, 1)])


def _s_body(ea_ref, eb_ref, out_ref):
    for v in range(V):
        out_ref[v] = jnp.sum(ea_ref[v, :]) + jnp.sum(eb_ref[v, :])


_s_tc = pl.pallas_call(
    _s_body,
    in_specs=[
        pl.BlockSpec((V, V), lambda: (0, 0)),
        pl.BlockSpec((V, V), lambda: (0, 0)),
    ],
    out_specs=pl.BlockSpec(memory_space=pltpu.SMEM),
    out_shape=jax.ShapeDtypeStruct((V,), jnp.float32),
)

TC_ROWS = SEQ - SEQ_SC
TC_COLS = B // 8


def _tc_body(x_ref, s_ref, out_ref, vacc_ref):
    c = pl.program_id(0)

    @pl.when(c == 0)
    def _init():
        vacc_ref[...] = jnp.zeros((8, 128), jnp.float32)

    xb = x_ref[...]
    val = jnp.full(xb.shape, s_ref[V - 1], jnp.float32)
    for v in range(V - 2, -1, -1):
        val = jnp.where(xb == v, s_ref[v], val)
    acc = vacc_ref[...]
    for i in range(TC_ROWS // 8):
        for k in range(TC_COLS // 128):
            acc = acc + val[i * 8:(i + 1) * 8, k * 128:(k + 1) * 128]
    vacc_ref[...] = acc

    @pl.when(c == 7)
    def _fini():
        out_ref[0, 0] = jnp.sum(vacc_ref[...])


_tc_sum = pl.pallas_call(
    _tc_body,
    grid=(8,),
    in_specs=[
        pl.BlockSpec((TC_ROWS, TC_COLS), lambda c: (SEQ_SC // TC_ROWS, c)),
        pl.BlockSpec(memory_space=pltpu.SMEM),
    ],
    out_specs=pl.BlockSpec(memory_space=pltpu.SMEM),
    out_shape=jax.ShapeDtypeStruct((1, 1), jnp.float32),
    scratch_shapes=[pltpu.VMEM((8, 128), jnp.float32)],
)


def kernel(x, emb, emb2):
    xt = x.astype(jnp.int32).T
    partials = _sc_sum(xt, emb, emb2)
    s10 = _s_tc(emb, emb2)
    tc_part = _tc_sum(xt, s10)
    return jnp.sum(partials) + tc_part[0, 0]


# speedup vs baseline: 1.2351x; 1.0407x over previous
import functools

import jax
import jax.numpy as jnp
from jax import lax
from jax.experimental import pallas as pl
from jax.experimental.pallas import tpu as pltpu
from jax.experimental.pallas import tpu_sc as plsc

L = 16
NC = 2
NS = 16
NW = NC * NS
V = 10

B, SEQ = 16384, 200
COLS_W = B // NW
RCHUNK = 24
SEQ_SC = 120
NCHUNK = SEQ_SC // RCHUNK
VROW = COLS_W // L
PAIRS = VROW // 2
TC_BR = (SEQ - SEQ_SC) // 8
TC_R0 = SEQ_SC // 8


@functools.partial(
    pl.kernel,
    out_type=jax.ShapeDtypeStruct((NW, L), jnp.float32),
    mesh=plsc.VectorSubcoreMesh(core_axis_name="c", subcore_axis_name="s"),
    compiler_params=pltpu.CompilerParams(needs_layout_passes=False),
    scratch_types=[
        pltpu.VMEM((RCHUNK, COLS_W), jnp.int32),
        pltpu.VMEM((RCHUNK, COLS_W), jnp.int32),
        pltpu.VMEM((V, V), jnp.float32),
        pltpu.VMEM((V, V), jnp.float32),
        pltpu.VMEM((L * L,), jnp.float32),
        pltpu.VMEM((1, L), jnp.float32),
        pltpu.SemaphoreType.DMA,
        pltpu.SemaphoreType.DMA,
    ],
)
def _sc_sum(xt_hbm, ea_hbm, eb_hbm, out_hbm,
            buf0, buf1, tab_a, tab_b, pair, acc_ref, sem0, sem1):
    cid = lax.axis_index("c")
    sid = lax.axis_index("s")
    wid = sid * NC + cid
    col0 = wid * COLS_W

    pltpu.sync_copy(ea_hbm, tab_a)
    pltpu.sync_copy(eb_hbm, tab_b)
    rows = lax.iota(jnp.int32, L)
    keep = rows < V
    zeros = jnp.zeros((L,), jnp.float32)
    s = zeros
    for k in range(V):
        col = jnp.full((L,), k, jnp.int32)
        s = s + plsc.load_gather(tab_a, [rows, col], mask=keep)
        s = s + plsc.load_gather(tab_b, [rows, col], mask=keep)
    s = jnp.where(keep, s, zeros)

    for a in range(V):
        pair[pl.ds(a * L, L)] = s[a] + s

    bufs = (buf0, buf1)
    sems = (sem0, sem1)

    def dma(c, buf, sem):
        return pltpu.make_async_copy(
            xt_hbm.at[pl.ds(c * RCHUNK, RCHUNK), pl.ds(col0, COLS_W)],
            buf, sem)

    dma(0, buf0, sem0).start()
    acc = zeros
    for c in range(NCHUNK):
        buf, sem = bufs[c % 2], sems[c % 2]
        if c + 1 < NCHUNK:
            dma(c + 1, bufs[(c + 1) % 2], sems[(c + 1) % 2]).start()
        dma(c, buf, sem).wait()

        def body(r, a, buf=buf):
            for j in range(PAIRS):
                ia = buf[r, pl.ds(2 * j * L, L)]
                ib = buf[r, pl.ds((2 * j + 1) * L, L)]
                a = a + plsc.load_gather(pair, [ia * L + ib])
            return a

        acc = lax.fori_loop(0, RCHUNK, body, acc)

    acc_ref[0, :] = acc
    pltpu.sync_copy(acc_ref, out_hbm.at[pl.ds(wid, 1)])


def _s_body(ea_ref, eb_ref, out_ref):
    for v in range(V):
        out_ref[v] = jnp.sum(ea_ref[v, :]) + jnp.sum(eb_ref[v, :])


_s_tc = pl.pallas_call(
    _s_body,
    in_specs=[
        pl.BlockSpec((V, V), lambda: (0, 0)),
        pl.BlockSpec((V, V), lambda: (0, 0)),
    ],
    out_specs=pl.BlockSpec(memory_space=pltpu.SMEM),
    out_shape=jax.ShapeDtypeStruct((V,), jnp.float32),
)

TC_ROWS = 40
TC_BR2 = (SEQ - SEQ_SC) // TC_ROWS
TC_COLS = B // 8


def _tc_body(x_ref, s_ref, out_ref, vacc_ref):
    r = pl.program_id(0)
    c = pl.program_id(1)

    @pl.when((r == 0) & (c == 0))
    def _init():
        vacc_ref[...] = jnp.zeros((8, 128), jnp.float32)

    xb = x_ref[...]
    val = jnp.full(xb.shape, s_ref[V - 1], jnp.float32)
    for v in range(V - 2, -1, -1):
        val = jnp.where(xb == v, s_ref[v], val)
    acc = vacc_ref[...]
    for i in range(TC_ROWS // 8):
        for k in range(TC_COLS // 128):
            acc = acc + val[i * 8:(i + 1) * 8, k * 128:(k + 1) * 128]
    vacc_ref[...] = acc

    @pl.when((r == TC_BR2 - 1) & (c == 7))
    def _fini():
        out_ref[0, 0] = jnp.sum(vacc_ref[...])


_tc_sum = pl.pallas_call(
    _tc_body,
    grid=(TC_BR2, 8),
    in_specs=[
        pl.BlockSpec((TC_ROWS, TC_COLS),
                     lambda r, c: (SEQ_SC // TC_ROWS + r, c)),
        pl.BlockSpec(memory_space=pltpu.SMEM),
    ],
    out_specs=pl.BlockSpec(memory_space=pltpu.SMEM),
    out_shape=jax.ShapeDtypeStruct((1, 1), jnp.float32),
    scratch_shapes=[pltpu.VMEM((8, 128), jnp.float32)],
)


def kernel(x, emb, emb2):
    xt = x.astype(jnp.int32).T
    partials = _sc_sum(xt, emb, emb2)
    s10 = _s_tc(emb, emb2)
    tc_part = _tc_sum(xt, s10)
    return jnp.sum(partials) + tc_part[0, 0]


# speedup vs baseline: 1.2355x; 1.0003x over previous
import functools

import jax
import jax.numpy as jnp
from jax import lax
from jax.experimental import pallas as pl
from jax.experimental.pallas import tpu as pltpu
from jax.experimental.pallas import tpu_sc as plsc

L = 16
NC = 2
NS = 16
NW = NC * NS
V = 10

B, SEQ = 16384, 200
COLS_W = B // NW
RCHUNK = 24
SEQ_SC = 120
NCHUNK = SEQ_SC // RCHUNK
VROW = COLS_W // L
PAIRS = VROW // 2
TC_BR = (SEQ - SEQ_SC) // 8
TC_R0 = SEQ_SC // 8


@functools.partial(
    pl.kernel,
    out_type=jax.ShapeDtypeStruct((NW, L), jnp.float32),
    mesh=plsc.VectorSubcoreMesh(core_axis_name="c", subcore_axis_name="s"),
    compiler_params=pltpu.CompilerParams(needs_layout_passes=False),
    scratch_types=[
        pltpu.VMEM((RCHUNK, COLS_W), jnp.int32),
        pltpu.VMEM((RCHUNK, COLS_W), jnp.int32),
        pltpu.VMEM((V, V), jnp.float32),
        pltpu.VMEM((V, V), jnp.float32),
        pltpu.VMEM((L * L,), jnp.float32),
        pltpu.VMEM((1, L), jnp.float32),
        pltpu.SemaphoreType.DMA,
        pltpu.SemaphoreType.DMA,
    ],
)
def _sc_sum(xt_hbm, ea_hbm, eb_hbm, out_hbm,
            buf0, buf1, tab_a, tab_b, pair, acc_ref, sem0, sem1):
    cid = lax.axis_index("c")
    sid = lax.axis_index("s")
    wid = sid * NC + cid
    col0 = wid * COLS_W

    pltpu.sync_copy(ea_hbm, tab_a)
    pltpu.sync_copy(eb_hbm, tab_b)
    rows = lax.iota(jnp.int32, L)
    keep = rows < V
    zeros = jnp.zeros((L,), jnp.float32)
    s = zeros
    for k in range(V):
        col = jnp.full((L,), k, jnp.int32)
        s = s + plsc.load_gather(tab_a, [rows, col], mask=keep)
        s = s + plsc.load_gather(tab_b, [rows, col], mask=keep)
    s = jnp.where(keep, s, zeros)

    for a in range(V):
        pair[pl.ds(a * L, L)] = s[a] + s

    bufs = (buf0, buf1)
    sems = (sem0, sem1)

    def dma(c, buf, sem):
        return pltpu.make_async_copy(
            xt_hbm.at[pl.ds(c * RCHUNK, RCHUNK), pl.ds(col0, COLS_W)],
            buf, sem)

    dma(0, buf0, sem0).start()
    acc = zeros
    for c in range(NCHUNK):
        buf, sem = bufs[c % 2], sems[c % 2]
        if c + 1 < NCHUNK:
            dma(c + 1, bufs[(c + 1) % 2], sems[(c + 1) % 2]).start()
        dma(c, buf, sem).wait()

        def body(r, a, buf=buf):
            for j in range(PAIRS):
                ia = buf[r, pl.ds(2 * j * L, L)]
                ib = buf[r, pl.ds((2 * j + 1) * L, L)]
                a = a + plsc.load_gather(pair, [ia * L + ib])
            return a

        acc = lax.fori_loop(0, RCHUNK, body, acc)

    acc_ref[0, :] = acc
    pltpu.sync_copy(acc_ref, out_hbm.at[pl.ds(wid, 1)])


TC_ROWS = 40
TC_BR2 = (SEQ - SEQ_SC) // TC_ROWS
TC_COLS = B // 8


def _tc_body(x_ref, ea_ref, eb_ref, out_ref, s_ref, vacc_ref):
    r = pl.program_id(0)
    c = pl.program_id(1)

    @pl.when((r == 0) & (c == 0))
    def _init():
        vacc_ref[...] = jnp.zeros((8, 128), jnp.float32)
        for v in range(V):
            sv = ea_ref[v, 0] + eb_ref[v, 0]
            for k in range(1, V):
                sv = sv + ea_ref[v, k] + eb_ref[v, k]
            s_ref[v] = sv

    xb = x_ref[...]
    val = jnp.full(xb.shape, s_ref[V - 1], jnp.float32)
    for v in range(V - 2, -1, -1):
        val = jnp.where(xb == v, s_ref[v], val)
    acc = vacc_ref[...]
    for i in range(TC_ROWS // 8):
        for k in range(TC_COLS // 128):
            acc = acc + val[i * 8:(i + 1) * 8, k * 128:(k + 1) * 128]
    vacc_ref[...] = acc

    @pl.when((r == TC_BR2 - 1) & (c == 7))
    def _fini():
        out_ref[0, 0] = jnp.sum(vacc_ref[...])


_tc_sum = pl.pallas_call(
    _tc_body,
    grid=(TC_BR2, 8),
    in_specs=[
        pl.BlockSpec((TC_ROWS, TC_COLS),
                     lambda r, c: (SEQ_SC // TC_ROWS + r, c)),
        pl.BlockSpec(memory_space=pltpu.SMEM),
        pl.BlockSpec(memory_space=pltpu.SMEM),
    ],
    out_specs=pl.BlockSpec(memory_space=pltpu.SMEM),
    out_shape=jax.ShapeDtypeStruct((1, 1), jnp.float32),
    scratch_shapes=[pltpu.SMEM((V,), jnp.float32),
                    pltpu.VMEM((8, 128), jnp.float32)],
)


def kernel(x, emb, emb2):
    xt = x.astype(jnp.int32).T
    partials = _sc_sum(xt, emb, emb2)
    tc_part = _tc_sum(xt, emb, emb2)
    return jnp.sum(partials) + tc_part[0, 0]
